# baseline (speedup 1.0000x reference)
import functools

import jax
import jax.numpy as jnp
from jax import lax
from jax.experimental import pallas as pl
from jax.experimental.pallas import tpu as pltpu
from jax.experimental.pallas import tpu_sc as plsc

LANES = 16


@functools.lru_cache(maxsize=None)
def _build(B, S, V, D, TV):
    info = plsc.get_sparse_core_info()
    NC, NS = info.num_cores, info.num_subcores
    NW = NC * NS
    assert S % NW == 0
    SEQW = S // NW
    SEQCH = 8
    assert SEQW % SEQCH == 0
    NCHUNK = SEQW // SEQCH
    DCH = D // LANES

    mesh = plsc.VectorSubcoreMesh(core_axis_name="c", subcore_axis_name="s")

    bcast_dnums = lax.GatherDimensionNumbers(
        offset_dims=(), collapsed_slice_dims=(0,), start_index_map=(0,))

    @functools.partial(
        pl.kernel,
        mesh=mesh,
        out_type=jax.ShapeDtypeStruct((B, S, D), jnp.float32),
        scratch_types=[
            pltpu.VMEM((B, SEQW), jnp.int32),
            pltpu.VMEM((B, SEQW + LANES), jnp.int32),
            pltpu.VMEM((B * SEQCH, D), jnp.float32),
            pltpu.VMEM((B * SEQCH, D), jnp.float32),
            pltpu.VMEM((B * SEQCH, D), jnp.float32),
            pltpu.VMEM((SEQCH, D), jnp.float32),
            pltpu.VMEM((SEQCH, D), jnp.float32),
            pltpu.VMEM((TV, D), jnp.float32),
            pltpu.VMEM((D,), jnp.float32),
            pltpu.SemaphoreType.DMA,
            pltpu.SemaphoreType.DMA,
            pltpu.SemaphoreType.DMA,
            pltpu.SemaphoreType.DMA,
            pltpu.SemaphoreType.DMA,
            pltpu.SemaphoreType.DMA,
            pltpu.SemaphoreType.DMA,
            pltpu.SemaphoreType.DMA,
            pltpu.SemaphoreType.DMA,
        ],
    )
    def emb(ids_hbm, tt_hbm, table_hbm, seg_hbm, pe_hbm, out_hbm,
            idx2d, tt2d, g0, g1, g2, p0, p1, seg_v, dlt_v,
            sg0, sg1, sg2, sp0, sp1, so0, so1, so2, s_setup):
        gbuf = (g0, g1, g2)
        pbuf = (p0, p1)
        sg = (sg0, sg1, sg2)
        sp = (sp0, sp1)
        so = (so0, so1, so2)

        wid = lax.axis_index("s") * NC + lax.axis_index("c")
        sq0 = wid * SEQW

        setup = []
        for b in range(B):
            setup.append(pltpu.make_async_copy(
                ids_hbm.at[b, pl.ds(sq0, SEQW)], idx2d.at[b], s_setup))
            setup.append(pltpu.make_async_copy(
                tt_hbm.at[b, pl.ds(sq0, SEQW)],
                tt2d.at[b, pl.ds(0, SEQW)], s_setup))
        setup.append(pltpu.make_async_copy(seg_hbm, seg_v, s_setup))
        for cp in setup:
            cp.start()

        def p_copy(c):
            return pltpu.make_async_copy(
                pe_hbm.at[pl.ds(sq0 + c * SEQCH, SEQCH)], pbuf[c % 2],
                sp[c % 2])

        p_copy(0).start()
        p_copy(1).start()

        for cp in setup:
            cp.wait()

        def g_copies(c):
            r = c % 3
            return [
                pltpu.make_async_copy(
                    table_hbm.at[idx2d.at[b, pl.ds(c * SEQCH, SEQCH)]],
                    gbuf[r].at[pl.ds(b * SEQCH, SEQCH)], sg[r])
                for b in range(B)
            ]

        HALF = SEQCH // 2

        def o_copies_half(c, h):
            r = c % 3
            return [
                pltpu.make_async_copy(
                    gbuf[r].at[pl.ds(b * SEQCH + h * HALF, HALF)],
                    out_hbm.at[b, pl.ds(sq0 + c * SEQCH + h * HALF, HALF)],
                    so[r])
                for b in range(B)
            ]

        def o_copies(c):
            return o_copies_half(c, 0) + o_copies_half(c, 1)

        for cp in g_copies(0):
            cp.start()

        def dlt(j, _):
            sl = pl.ds(j * LANES, LANES)
            dlt_v[sl] = seg_v[1, sl] - seg_v[0, sl]
            return 0

        lax.fori_loop(0, DCH, dlt, 0)

        for c in range(NCHUNK):
            r = c % 3
            if c >= 2:
                for cp in o_copies(c - 2):
                    cp.wait()
            if c + 1 < NCHUNK:
                for cp in g_copies(c + 1):
                    cp.start()
            for cp in g_copies(c):
                cp.wait()
            p_copy(c).wait()

            gb = gbuf[r]
            pb = pbuf[c % 2]
            ttvs = [tt2d[b, pl.ds(c * SEQCH, LANES)].astype(jnp.float32)
                    for b in range(B)]

            def half_loop(h):
                def dloop(j, _):
                    sl = pl.ds(j * LANES, LANES)
                    s0v = seg_v[0, sl]
                    dv = dlt_v[sl]
                    ks = range(h * HALF, (h + 1) * HALF)
                    pek = {k: pb[k, sl] + s0v for k in ks}
                    for b in range(B):
                        for k in ks:
                            f = lax.gather(
                                ttvs[b], jnp.full((LANES, 1), k, jnp.int32),
                                bcast_dnums, (1,),
                                mode=lax.GatherScatterMode.PROMISE_IN_BOUNDS)
                            i = b * SEQCH + k
                            gb[i, sl] = gb[i, sl] + pek[k] + f * dv
                    return 0

                lax.fori_loop(0, DCH, dloop, 0)

            for h in range(2):
                half_loop(h)
                for cp in o_copies_half(c, h):
                    cp.start()
            if c + 2 < NCHUNK:
                p_copy(c + 2).start()

        for c in (NCHUNK - 2, NCHUNK - 1):
            for cp in o_copies(c):
                cp.wait()

    return emb


def kernel(input_ids, token_type_ids, token_table, segment_table, pe):
    B, S = input_ids.shape
    V, D = token_table.shape
    TV = segment_table.shape[0]
    ids = input_ids if input_ids.dtype == jnp.int32 else (
        input_ids.astype(jnp.int32))
    tt = token_type_ids if token_type_ids.dtype == jnp.int32 else (
        token_type_ids.astype(jnp.int32))
    emb = _build(B, S, V, D, TV)
    return emb(ids, tt, token_table, segment_table, pe)
